# baseline (speedup 1.0000x reference)
import functools

import jax
import jax.numpy as jnp
from jax import lax
from jax.experimental import pallas as pl
from jax.experimental.pallas import tpu as pltpu
from jax.experimental.pallas import tpu_sc as plsc

DIM = 128
NC = 2
NS = 16
NW = NC * NS
RPC = 2
NB = 4


def _body(hist, nch, idx_hbm, table_hbm, out_hbm, idx_v, rows_v, gsem, osem):
    wid = lax.axis_index("s") * NC + lax.axis_index("c")
    row0 = wid * (nch * RPC)
    pltpu.sync_copy(idx_hbm.at[wid], idx_v)

    def gather(j, b):
        pltpu.make_async_copy(
            table_hbm.at[idx_v.at[j]], rows_v.at[b], gsem.at[b]
        ).start()

    def outcopy(j, b, r):
        return pltpu.make_async_copy(
            rows_v.at[b].at[pl.ds(r * hist, hist)],
            out_hbm.at[row0 + j * RPC + r],
            osem.at[b],
        )

    for b in range(NB):
        gather(b, b)

    def step(j, carry):
        b = lax.rem(j, NB)
        pltpu.make_async_copy(
            table_hbm.at[idx_v.at[j]], rows_v.at[b], gsem.at[b]
        ).wait()
        for r in range(RPC):
            outcopy(j, b, r).start()

        @pl.when(j + NB < nch)
        def _():
            for r in range(RPC):
                outcopy(j, b, r).wait()
            gather(j + NB, b)

        return carry

    lax.fori_loop(0, nch, step, 0)
    for i in range(NB):
        j = nch - NB + i
        for r in range(RPC):
            outcopy(j, lax.rem(jnp.int32(j), NB), r).wait()


def kernel(indices, table):
    batch, hist = indices.shape
    assert batch % (NW * RPC) == 0
    nch = batch // (NW * RPC)
    assert nch >= NB
    ipc = RPC * hist
    assert ipc <= 128
    idx = indices.reshape(NW, nch, ipc).astype(jnp.int32)

    mesh = plsc.VectorSubcoreMesh(core_axis_name="c", subcore_axis_name="s")
    k = functools.partial(
        pl.kernel,
        mesh=mesh,
        compiler_params=pltpu.CompilerParams(use_tc_tiling_on_sc=True),
        out_type=jax.ShapeDtypeStruct((batch, hist, DIM), jnp.float32),
        scratch_types=[
            pltpu.VMEM((nch, ipc), jnp.int32),
            pltpu.VMEM((NB, ipc, DIM), jnp.float32),
            pltpu.SemaphoreType.DMA((NB,)),
            pltpu.SemaphoreType.DMA((NB,)),
        ],
    )(functools.partial(_body, hist, nch))
    return k(idx, table)


# speedup vs baseline: 1.7598x; 1.7598x over previous
import functools

import jax
import jax.numpy as jnp
from jax import lax
from jax.experimental import pallas as pl
from jax.experimental.pallas import tpu as pltpu
from jax.experimental.pallas import tpu_sc as plsc

DIM = 128
NC = 2
NS = 16
NW = NC * NS
CH = 128
NB = 4


def _body(nch, idx_hbm, table_hbm, out_hbm, idx_v, rows_v, gsem, osem):
    wid = lax.axis_index("s") * NC + lax.axis_index("c")
    base = wid * (nch * CH)
    pltpu.sync_copy(idx_hbm.at[wid], idx_v)

    def gather(j, b):
        pltpu.make_async_copy(
            table_hbm.at[idx_v.at[j]], rows_v.at[b], gsem.at[b]
        ).start()

    def outcopy(j, b):
        return pltpu.make_async_copy(
            rows_v.at[b], out_hbm.at[pl.ds(base + j * CH, CH)], osem.at[b]
        )

    for b in range(NB):
        gather(b, b)

    def step(j, carry):
        b = lax.rem(j, NB)
        pltpu.make_async_copy(
            table_hbm.at[idx_v.at[j]], rows_v.at[b], gsem.at[b]
        ).wait()
        outcopy(j, b).start()

        @pl.when(j + NB < nch)
        def _():
            outcopy(j, b).wait()
            gather(j + NB, b)

        return carry

    lax.fori_loop(0, nch, step, 0)
    for i in range(NB):
        j = nch - NB + i
        outcopy(j, lax.rem(jnp.int32(j), NB)).wait()


def kernel(indices, table):
    batch, hist = indices.shape
    n = batch * hist
    assert n % (NW * CH) == 0
    nch = n // (NW * CH)
    assert nch >= NB
    idx = jnp.transpose(indices).reshape(NW, nch, CH).astype(jnp.int32)

    mesh = plsc.VectorSubcoreMesh(core_axis_name="c", subcore_axis_name="s")
    k = functools.partial(
        pl.kernel,
        mesh=mesh,
        out_type=jax.ShapeDtypeStruct((n, DIM), jnp.float32),
        scratch_types=[
            pltpu.VMEM((nch, CH), jnp.int32),
            pltpu.VMEM((NB, CH, DIM), jnp.float32),
            pltpu.SemaphoreType.DMA((NB,)),
            pltpu.SemaphoreType.DMA((NB,)),
        ],
    )(functools.partial(_body, nch))
    out = k(idx, table)
    return jnp.transpose(out.reshape(hist, batch, DIM), (1, 0, 2))


# speedup vs baseline: 1.7717x; 1.0068x over previous
import functools

import jax
import jax.numpy as jnp
from jax import lax
from jax.experimental import pallas as pl
from jax.experimental.pallas import tpu as pltpu
from jax.experimental.pallas import tpu_sc as plsc

DIM = 128
NC = 2
NS = 16
NW = NC * NS
CH = 128
NB = 6


def _body(nch, idx_hbm, table_hbm, out_hbm, idx_v, rows_v, gsem, osem):
    wid = lax.axis_index("s") * NC + lax.axis_index("c")
    base = wid * (nch * CH)
    pltpu.sync_copy(idx_hbm.at[wid], idx_v)

    def gather(j, b):
        pltpu.make_async_copy(
            table_hbm.at[idx_v.at[j]], rows_v.at[b], gsem.at[b]
        ).start()

    def outcopy(j, b):
        return pltpu.make_async_copy(
            rows_v.at[b], out_hbm.at[pl.ds(base + j * CH, CH)], osem.at[b]
        )

    for b in range(NB):
        gather(b, b)

    def step(j, carry):
        b = lax.rem(j, NB)
        pltpu.make_async_copy(
            table_hbm.at[idx_v.at[j]], rows_v.at[b], gsem.at[b]
        ).wait()
        outcopy(j, b).start()

        @pl.when(j + NB < nch)
        def _():
            outcopy(j, b).wait()
            gather(j + NB, b)

        return carry

    lax.fori_loop(0, nch, step, 0)
    for i in range(NB):
        j = nch - NB + i
        outcopy(j, lax.rem(jnp.int32(j), NB)).wait()


def kernel(indices, table):
    batch, hist = indices.shape
    n = batch * hist
    assert n % (NW * CH) == 0
    nch = n // (NW * CH)
    assert nch >= NB
    idx = jnp.transpose(indices).reshape(NW, nch, CH).astype(jnp.int32)

    mesh = plsc.VectorSubcoreMesh(core_axis_name="c", subcore_axis_name="s")
    k = functools.partial(
        pl.kernel,
        mesh=mesh,
        out_type=jax.ShapeDtypeStruct((n, DIM), jnp.float32),
        scratch_types=[
            pltpu.VMEM((nch, CH), jnp.int32),
            pltpu.VMEM((NB, CH, DIM), jnp.float32),
            pltpu.SemaphoreType.DMA((NB,)),
            pltpu.SemaphoreType.DMA((NB,)),
        ],
    )(functools.partial(_body, nch))
    out = k(idx, table)
    return jnp.transpose(out.reshape(hist, batch, DIM), (1, 0, 2))
